# baseline (speedup 1.0000x reference)
import jax
import jax.numpy as jnp
from jax.experimental import pallas as pl
from jax.experimental.pallas import tpu as pltpu

_R, _N, _D, _C = 64, 512, 512, 2
_PATCH_TOPK, _REGION_TOPK = 100, 10
_AH = 128


def _topk_mean(sim, k, iters=48):
    lo = jnp.min(sim, axis=0, keepdims=True) - 1.0
    hi = jnp.max(sim, axis=0, keepdims=True) + 1.0

    def body(_, carry):
        lo, hi = carry
        mid = 0.5 * (lo + hi)
        cnt = jnp.sum((sim >= mid).astype(jnp.float32), axis=0, keepdims=True)
        ge = cnt >= k
        return jnp.where(ge, mid, lo), jnp.where(ge, hi, mid)

    lo, hi = jax.lax.fori_loop(0, iters, body, (lo, hi))
    t = lo
    gt = sim > t
    s = jnp.sum(jnp.where(gt, sim, 0.0), axis=0, keepdims=True)
    cnt = jnp.sum(gt.astype(jnp.float32), axis=0, keepdims=True)
    return (s + (k - cnt) * t) / k


def _stage1_body(x_ref, tf_ref, w1_ref, b1_ref, w2_ref, b2_ref, fw_ref, fb_ref,
                 aw1_ref, ab1_ref, aw2_ref, ab2_ref, ls_ref, rm_ref, sim_ref):
    xr = x_ref[0]
    h = jnp.maximum(jnp.dot(xr, w1_ref[...], preferred_element_type=jnp.float32)
                    + b1_ref[...], 0.0)
    patch = jnp.dot(h, w2_ref[...], preferred_element_type=jnp.float32) + b2_ref[...]
    feat = jnp.maximum(jnp.dot(patch, fw_ref[...], preferred_element_type=jnp.float32)
                       + fb_ref[...], 0.0)
    t = jnp.tanh(jnp.dot(feat, aw1_ref[...], preferred_element_type=jnp.float32)
                 + ab1_ref[...])
    a = jnp.dot(t, aw2_ref[...], preferred_element_type=jnp.float32) + ab2_ref[...]
    a = a - jnp.max(a)
    e = jnp.exp(a)
    w = e / jnp.sum(e)
    rm_ref[0] = jnp.dot(w.T, feat, preferred_element_type=jnp.float32)

    tf = tf_ref[...]
    tn = tf / (jnp.sqrt(jnp.sum(tf * tf, axis=1, keepdims=True)) + 1e-8)
    scale = jnp.exp(ls_ref[0, 0])
    s = jnp.dot(patch, tn.T, preferred_element_type=jnp.float32)
    pn = jnp.sqrt(jnp.sum(patch * patch, axis=1, keepdims=True))
    sim_ref[0] = scale * (s / (pn + 1e-8))


def _stage2_body(rm_ref, sim_ref, tf_ref, rw1_ref, rb1_ref, rw2_ref, rb2_ref,
                 sw1_ref, sb1_ref, sw2_ref, sb2_ref, fw_ref, fb_ref,
                 aw1_ref, ab1_ref, aw2_ref, ab2_ref, ls_ref, out_ref):
    rm = rm_ref[...]
    h = jnp.maximum(jnp.dot(rm, rw1_ref[...], preferred_element_type=jnp.float32)
                    + rb1_ref[...], 0.0)
    region = jnp.dot(h, rw2_ref[...], preferred_element_type=jnp.float32) + rb2_ref[...]
    feat = jnp.maximum(jnp.dot(region, fw_ref[...], preferred_element_type=jnp.float32)
                       + fb_ref[...], 0.0)
    t = jnp.tanh(jnp.dot(feat, aw1_ref[...], preferred_element_type=jnp.float32)
                 + ab1_ref[...])
    a = jnp.dot(t, aw2_ref[...], preferred_element_type=jnp.float32) + ab2_ref[...]
    a = a - jnp.max(a)
    e = jnp.exp(a)
    w = e / jnp.sum(e)
    slide_m = jnp.dot(w.T, feat, preferred_element_type=jnp.float32)
    hs = jnp.maximum(jnp.dot(slide_m, sw1_ref[...], preferred_element_type=jnp.float32)
                     + sb1_ref[...], 0.0)
    slide = jnp.dot(hs, sw2_ref[...], preferred_element_type=jnp.float32) + sb2_ref[...]

    tf = tf_ref[...]
    tn = tf / (jnp.sqrt(jnp.sum(tf * tf, axis=1, keepdims=True)) + 1e-8)
    scale = jnp.exp(ls_ref[0, 0])

    sn = jnp.sqrt(jnp.sum(slide * slide, axis=1, keepdims=True))
    slide_logits = scale * jnp.dot(slide / (sn + 1e-8), tn.T,
                                   preferred_element_type=jnp.float32)

    rn = jnp.sqrt(jnp.sum(region * region, axis=1, keepdims=True))
    rsim = scale * (jnp.dot(region, tn.T, preferred_element_type=jnp.float32)
                    / (rn + 1e-8))
    region_logits = _topk_mean(rsim, _REGION_TOPK)

    psim = sim_ref[...].reshape(_R * _N, _C)
    patch_logits = _topk_mean(psim, _PATCH_TOPK)

    out_ref[...] = slide_logits + region_logits + patch_logits


def kernel(x, txt_feats, pp_w1, pp_b1, pp_w2, pp_b2, rp_w1, rp_b1, rp_w2, rp_b2,
           sp_w1, sp_b1, sp_w2, sp_b2, p2r_fw, p2r_fb, p2r_aw1, p2r_ab1, p2r_aw2,
           p2r_ab2, r2s_fw, r2s_fb, r2s_aw1, r2s_ab1, r2s_aw2, r2s_ab2, logit_scale):
    f32 = jnp.float32
    ls = logit_scale.reshape(1, 1)

    full = lambda shape: pl.BlockSpec(shape, lambda r: tuple(0 for _ in shape))
    rm, sim = pl.pallas_call(
        _stage1_body,
        grid=(_R,),
        in_specs=[
            pl.BlockSpec((1, _N, _D), lambda r: (r, 0, 0)),
            full((_C, _D)),
            full((_D, _D)), full((1, _D)),
            full((_D, _D)), full((1, _D)),
            full((_D, _D)), full((1, _D)),
            full((_D, _AH)), full((1, _AH)),
            full((_AH, 1)), full((1, 1)),
            full((1, 1)),
        ],
        out_specs=[
            pl.BlockSpec((1, 1, _D), lambda r: (r, 0, 0)),
            pl.BlockSpec((1, _N, _C), lambda r: (r, 0, 0)),
        ],
        out_shape=[
            jax.ShapeDtypeStruct((_R, 1, _D), f32),
            jax.ShapeDtypeStruct((_R, _N, _C), f32),
        ],
        compiler_params=pltpu.CompilerParams(
            dimension_semantics=("arbitrary",),
        ),
    )(x, txt_feats, pp_w1, pp_b1.reshape(1, _D), pp_w2, pp_b2.reshape(1, _D),
      p2r_fw, p2r_fb.reshape(1, _D), p2r_aw1, p2r_ab1.reshape(1, _AH),
      p2r_aw2, p2r_ab2.reshape(1, 1), ls)

    out = pl.pallas_call(
        _stage2_body,
        out_shape=jax.ShapeDtypeStruct((1, _C), f32),
    )(rm.reshape(_R, _D), sim, txt_feats, rp_w1, rp_b1.reshape(1, _D), rp_w2, rp_b2.reshape(1, _D),
      sp_w1, sp_b1.reshape(1, _D), sp_w2, sp_b2.reshape(1, _D),
      r2s_fw, r2s_fb.reshape(1, _D), r2s_aw1, r2s_ab1.reshape(1, _AH),
      r2s_aw2, r2s_ab2.reshape(1, 1), ls)

    return out.reshape(_C)


# speedup vs baseline: 3.1324x; 3.1324x over previous
import jax
import jax.numpy as jnp
from jax.experimental import pallas as pl
from jax.experimental.pallas import tpu as pltpu

_R, _N, _D, _C = 64, 512, 512, 2
_PATCH_TOPK, _REGION_TOPK = 100, 10
_AH = 128
_BR = 4


def _topk_mean(sim, k, iters=42):
    c = sim.shape[0]
    red = tuple(range(1, sim.ndim))
    lo = jnp.min(sim, axis=red).reshape(c, *([1] * (sim.ndim - 1))) - 1.0
    hi = jnp.max(sim, axis=red).reshape(c, *([1] * (sim.ndim - 1))) + 1.0

    def body(_, carry):
        lo, hi = carry
        mid = 0.5 * (lo + hi)
        cnt = jnp.sum((sim >= mid).astype(jnp.float32), axis=red,
                      keepdims=True)
        ge = cnt >= k
        return jnp.where(ge, mid, lo), jnp.where(ge, hi, mid)

    lo, hi = jax.lax.fori_loop(0, iters, body, (lo, hi))
    t = lo
    gt = sim > t
    s = jnp.sum(jnp.where(gt, sim, 0.0), axis=red, keepdims=True)
    cnt = jnp.sum(gt.astype(jnp.float32), axis=red, keepdims=True)
    out = (s + (k - cnt) * t) / k
    return out.reshape(c, 1)


def _stage1_body(x_ref, tf_ref, w1_ref, b1_ref, w2_ref, b2_ref, fw_ref, fb_ref,
                 aw1_ref, ab1_ref, aw2_ref, ab2_ref, ls_ref, rm_ref, sim_ref):
    xr = x_ref[...].reshape(_BR * _N, _D)
    h = jnp.maximum(jnp.dot(xr, w1_ref[...], preferred_element_type=jnp.float32)
                    + b1_ref[...], 0.0)
    patch = jnp.dot(h, w2_ref[...], preferred_element_type=jnp.float32) + b2_ref[...]
    feat = jnp.maximum(jnp.dot(patch, fw_ref[...], preferred_element_type=jnp.float32)
                       + fb_ref[...], 0.0)
    t = jnp.tanh(jnp.dot(feat, aw1_ref[...], preferred_element_type=jnp.float32)
                 + ab1_ref[...])
    a = jnp.dot(t, aw2_ref[...], preferred_element_type=jnp.float32) + ab2_ref[...]
    a = a.reshape(_BR, _N)
    a = a - jnp.max(a, axis=1, keepdims=True)
    e = jnp.exp(a)
    w = e / jnp.sum(e, axis=1, keepdims=True)
    wf = w.reshape(_BR * _N, 1) * feat
    rm_ref[0] = jnp.sum(wf.reshape(_BR, _N, _D), axis=1)

    tf = tf_ref[...]
    tn = tf / (jnp.sqrt(jnp.sum(tf * tf, axis=1, keepdims=True)) + 1e-8)
    scale = jnp.exp(ls_ref[0, 0])
    s = jax.lax.dot_general(tn, patch, (((1,), (1,)), ((), ())),
                            preferred_element_type=jnp.float32)
    pn = jnp.sqrt(jnp.sum(patch * patch, axis=1, keepdims=True))
    inv = (1.0 / (pn + 1e-8)).reshape(1, _BR * _N)
    sim_ref[...] = scale * s * inv


def _stage2_body(rm_ref, sim_ref, tf_ref, rw1_ref, rb1_ref, rw2_ref, rb2_ref,
                 sw1_ref, sb1_ref, sw2_ref, sb2_ref, fw_ref, fb_ref,
                 aw1_ref, ab1_ref, aw2_ref, ab2_ref, ls_ref, out_ref):
    rm = rm_ref[...]
    h = jnp.maximum(jnp.dot(rm, rw1_ref[...], preferred_element_type=jnp.float32)
                    + rb1_ref[...], 0.0)
    region = jnp.dot(h, rw2_ref[...], preferred_element_type=jnp.float32) + rb2_ref[...]
    feat = jnp.maximum(jnp.dot(region, fw_ref[...], preferred_element_type=jnp.float32)
                       + fb_ref[...], 0.0)
    t = jnp.tanh(jnp.dot(feat, aw1_ref[...], preferred_element_type=jnp.float32)
                 + ab1_ref[...])
    a = jnp.dot(t, aw2_ref[...], preferred_element_type=jnp.float32) + ab2_ref[...]
    a = a - jnp.max(a)
    e = jnp.exp(a)
    w = e / jnp.sum(e)
    slide_m = jnp.dot(w.T, feat, preferred_element_type=jnp.float32)
    hs = jnp.maximum(jnp.dot(slide_m, sw1_ref[...], preferred_element_type=jnp.float32)
                     + sb1_ref[...], 0.0)
    slide = jnp.dot(hs, sw2_ref[...], preferred_element_type=jnp.float32) + sb2_ref[...]

    tf = tf_ref[...]
    tn = tf / (jnp.sqrt(jnp.sum(tf * tf, axis=1, keepdims=True)) + 1e-8)
    scale = jnp.exp(ls_ref[0, 0])

    sn = jnp.sqrt(jnp.sum(slide * slide, axis=1, keepdims=True))
    slide_logits = scale * jnp.dot(slide / (sn + 1e-8), tn.T,
                                   preferred_element_type=jnp.float32)

    rn = jnp.sqrt(jnp.sum(region * region, axis=1, keepdims=True))
    rsim = scale * (jax.lax.dot_general(tn, region, (((1,), (1,)), ((), ())),
                                        preferred_element_type=jnp.float32)
                    / (rn + 1e-8).reshape(1, _R))
    region_logits = _topk_mean(rsim, _REGION_TOPK)

    psim = sim_ref[...].reshape(_C, (_R * _N) // 128, 128)
    patch_logits = _topk_mean(psim, _PATCH_TOPK)

    out_ref[...] = slide_logits + region_logits.T + patch_logits.T


def kernel(x, txt_feats, pp_w1, pp_b1, pp_w2, pp_b2, rp_w1, rp_b1, rp_w2, rp_b2,
           sp_w1, sp_b1, sp_w2, sp_b2, p2r_fw, p2r_fb, p2r_aw1, p2r_ab1, p2r_aw2,
           p2r_ab2, r2s_fw, r2s_fb, r2s_aw1, r2s_ab1, r2s_aw2, r2s_ab2, logit_scale):
    f32 = jnp.float32
    ls = logit_scale.reshape(1, 1)

    full = lambda shape: pl.BlockSpec(shape, lambda r: tuple(0 for _ in shape))
    rm, sim = pl.pallas_call(
        _stage1_body,
        grid=(_R // _BR,),
        in_specs=[
            pl.BlockSpec((_BR, _N, _D), lambda r: (r, 0, 0)),
            full((_C, _D)),
            full((_D, _D)), full((1, _D)),
            full((_D, _D)), full((1, _D)),
            full((_D, _D)), full((1, _D)),
            full((_D, _AH)), full((1, _AH)),
            full((_AH, 1)), full((1, 1)),
            full((1, 1)),
        ],
        out_specs=[
            pl.BlockSpec((1, _BR, _D), lambda r: (r, 0, 0)),
            pl.BlockSpec((_C, _BR * _N), lambda r: (0, r)),
        ],
        out_shape=[
            jax.ShapeDtypeStruct((_R // _BR, _BR, _D), f32),
            jax.ShapeDtypeStruct((_C, _R * _N), f32),
        ],
        compiler_params=pltpu.CompilerParams(
            dimension_semantics=("parallel",),
        ),
    )(x, txt_feats, pp_w1, pp_b1.reshape(1, _D), pp_w2, pp_b2.reshape(1, _D),
      p2r_fw, p2r_fb.reshape(1, _D), p2r_aw1, p2r_ab1.reshape(1, _AH),
      p2r_aw2, p2r_ab2.reshape(1, 1), ls)

    out = pl.pallas_call(
        _stage2_body,
        out_shape=jax.ShapeDtypeStruct((1, _C), f32),
    )(rm.reshape(_R, _D), sim, txt_feats, rp_w1, rp_b1.reshape(1, _D), rp_w2, rp_b2.reshape(1, _D),
      sp_w1, sp_b1.reshape(1, _D), sp_w2, sp_b2.reshape(1, _D),
      r2s_fw, r2s_fb.reshape(1, _D), r2s_aw1, r2s_ab1.reshape(1, _AH),
      r2s_aw2, r2s_ab2.reshape(1, 1), ls)

    return out.reshape(_C)
